# Initial kernel scaffold; baseline (speedup 1.0000x reference)
import jax
import jax.numpy as jnp
from jax.experimental import pallas as pl


def kernel(pointcloud, params):
    raise NotImplementedError("write your pallas kernel here")


# speedup vs baseline: 10.9380x; 10.9380x over previous
import functools
import jax
import jax.numpy as jnp
import numpy as np
from jax.experimental import pallas as pl
from jax.experimental.pallas import tpu as pltpu

_BN = float(np.sqrt(1.0 + 1e-5))
_B = 8
_N1 = 4096
_M1 = 512
_M2 = 128
_NS = 64


def _fps_body(x_ref, ox_ref, oy_ref, oz_ref, dists_ref, far_ref, *, n):
    t = pl.program_id(0)

    @pl.when(t == 0)
    def _():
        dists_ref[...] = jnp.full((_B, n), 1e10, jnp.float32)
        far_ref[...] = jnp.zeros((_B, 1), jnp.int32)

    far = far_ref[...]
    lane = jax.lax.broadcasted_iota(jnp.int32, (_B, n), 1)
    sel = lane == far
    xs = x_ref[0]
    ys = x_ref[1]
    zs = x_ref[2]
    cx = jnp.sum(jnp.where(sel, xs, 0.0), axis=1, keepdims=True)
    cy = jnp.sum(jnp.where(sel, ys, 0.0), axis=1, keepdims=True)
    cz = jnp.sum(jnp.where(sel, zs, 0.0), axis=1, keepdims=True)
    ox_ref[0] = cx
    oy_ref[0] = cy
    oz_ref[0] = cz
    d = (xs - cx) ** 2 + (ys - cy) ** 2 + (zs - cz) ** 2
    dmin = jnp.minimum(dists_ref[...], d)
    dists_ref[...] = dmin
    m = jnp.max(dmin, axis=1, keepdims=True)
    nxt = jnp.min(jnp.where(dmin == m, lane, n), axis=1, keepdims=True)
    far_ref[...] = nxt.astype(jnp.int32)


def _fps(x3bn, npoint):
    n = x3bn.shape[2]
    out = jax.ShapeDtypeStruct((npoint, _B, 1), jnp.float32)
    cx, cy, cz = pl.pallas_call(
        functools.partial(_fps_body, n=n),
        grid=(npoint,),
        in_specs=[pl.BlockSpec((3, _B, n), lambda t: (0, 0, 0))],
        out_specs=[pl.BlockSpec((1, _B, 1), lambda t: (t, 0, 0))] * 3,
        out_shape=[out, out, out],
        scratch_shapes=[pltpu.VMEM((_B, n), jnp.float32),
                        pltpu.VMEM((_B, 1), jnp.int32)],
    )(x3bn)
    tr = lambda a: jnp.transpose(a[:, :, 0], (1, 0))
    return tr(cx), tr(cy), tr(cz)


def _bq_body(x_ref, c_ref, o_ref, *, rsq, n, tm):
    b = pl.program_id(0)
    X = x_ref[0]
    C = c_ref[...]
    dot = jnp.dot(C, X, preferred_element_type=jnp.float32)
    a2 = jnp.sum(C * C, axis=1, keepdims=True)
    b2 = jnp.sum(X * X, axis=0, keepdims=True)
    sq = (-2.0 * dot) + a2 + b2
    mask = (sq <= rsq).astype(jnp.int32)
    pos = mask
    k = 1
    while k < n:
        shifted = jnp.concatenate(
            [jnp.zeros((tm, k), jnp.int32), pos[:, : n - k]], axis=1)
        pos = pos + shifted
        k *= 2
    cols = []
    for j in range(_NS):
        cols.append(jnp.sum((pos <= j).astype(jnp.int32), axis=1, keepdims=True))
    idx = jnp.concatenate(cols, axis=1)
    first = idx[:, 0:1]
    idx = jnp.where(idx == n, jnp.broadcast_to(first, idx.shape), idx)
    idx = jnp.where(idx == n, 0, idx)
    o_ref[...] = idx + b * n


def _ball_query(xb3n, cflat, radius, tm):
    n = xb3n.shape[2]
    m_total = cflat.shape[0]
    mper = m_total // _B
    grid = (_B, mper // tm)
    return pl.pallas_call(
        functools.partial(_bq_body, rsq=radius * radius, n=n, tm=tm),
        grid=grid,
        in_specs=[
            pl.BlockSpec((1, 3, n), lambda b, m: (b, 0, 0)),
            pl.BlockSpec((tm, 3), lambda b, m, mper_t=mper // tm: (b * mper_t + m, 0)),
        ],
        out_specs=pl.BlockSpec((tm, _NS), lambda b, m, mper_t=mper // tm: (b * mper_t + m, 0)),
        out_shape=jax.ShapeDtypeStruct((m_total, _NS), jnp.int32),
    )(xb3n, cflat)


def _mlp_body(g_ref, c_ref, w1_ref, b1_ref, w2_ref, b2_ref, w3_ref, b3_ref,
              o_ref, *, tm):
    rows = tm * _NS
    rep = (jax.lax.broadcasted_iota(jnp.int32, (rows, tm), 0) // _NS ==
           jax.lax.broadcasted_iota(jnp.int32, (rows, tm), 1)).astype(jnp.float32)
    ce = jnp.dot(rep, c_ref[...], preferred_element_type=jnp.float32)
    g = g_ref[...] - ce
    h = jax.nn.relu(jnp.dot(g, w1_ref[...], preferred_element_type=jnp.float32)
                    + b1_ref[...])
    h = jax.nn.relu(jnp.dot(h, w2_ref[...], preferred_element_type=jnp.float32)
                    + b2_ref[...])
    h = jax.nn.relu(jnp.dot(h, w3_ref[...], preferred_element_type=jnp.float32)
                    + b3_ref[...])
    for m in range(tm):
        o_ref[m:m + 1, :] = jnp.max(h[m * _NS:(m + 1) * _NS, :], axis=0,
                                    keepdims=True)


def _group_mlp(gather_rows, cpad, ws, tm):
    m_total = cpad.shape[0]
    dpad = gather_rows.shape[1]
    (w1, b1), (w2, b2), (w3, b3) = ws
    cout = w3.shape[1]
    grid = (m_total // tm,)
    full = lambda a: pl.BlockSpec(a.shape, lambda i: tuple(0 for _ in a.shape))
    return pl.pallas_call(
        functools.partial(_mlp_body, tm=tm),
        grid=grid,
        in_specs=[
            pl.BlockSpec((tm * _NS, dpad), lambda i: (i, 0)),
            pl.BlockSpec((tm, dpad), lambda i: (i, 0)),
            full(w1), full(b1), full(w2), full(b2), full(w3), full(b3),
        ],
        out_specs=pl.BlockSpec((tm, cout), lambda i: (i, 0)),
        out_shape=jax.ShapeDtypeStruct((m_total, cout), jnp.float32),
    )(gather_rows, cpad, w1, b1, w2, b2, w3, b3)


def _head_body(t_ref, w1_ref, b1_ref, w2_ref, b2_ref, w3_ref, b3_ref,
               f1_ref, fb1_ref, f2_ref, fb2_ref, f3_ref, fb3_ref,
               o_ref, pool_ref):
    h = t_ref[...]
    h = jax.nn.relu(jnp.dot(h, w1_ref[...], preferred_element_type=jnp.float32)
                    + b1_ref[...])
    h = jax.nn.relu(jnp.dot(h, w2_ref[...], preferred_element_type=jnp.float32)
                    + b2_ref[...])
    h = jax.nn.relu(jnp.dot(h, w3_ref[...], preferred_element_type=jnp.float32)
                    + b3_ref[...])
    for b in range(_B):
        pool_ref[b:b + 1, :] = jnp.max(h[b * _M2:(b + 1) * _M2, :], axis=0,
                                       keepdims=True)
    x = pool_ref[...]
    x = jax.nn.relu(jnp.dot(x, f1_ref[...], preferred_element_type=jnp.float32)
                    + fb1_ref[...])
    x = jax.nn.relu(jnp.dot(x, f2_ref[...], preferred_element_type=jnp.float32)
                    + fb2_ref[...])
    o_ref[...] = jnp.dot(x, f3_ref[...], preferred_element_type=jnp.float32) + fb3_ref[...]


def _head(t3, ws, fcs):
    (w1, b1), (w2, b2), (w3, b3) = ws
    (f1, fb1), (f2, fb2), (f3, fb3) = fcs
    nc = f3.shape[1]
    args = (t3, w1, b1, w2, b2, w3, b3, f1, fb1, f2, fb2, f3, fb3)
    return pl.pallas_call(
        _head_body,
        out_shape=jax.ShapeDtypeStruct((_B, nc), jnp.float32),
        scratch_shapes=[pltpu.VMEM((_B, w3.shape[1]), jnp.float32)],
    )(*args)


def _gather_rows(table, gidx):
    return jnp.take(table, gidx, axis=0)


def _prep_sa(layers, dpad):
    out = []
    for i, (w, b) in enumerate(layers):
        wt = jnp.transpose(w) / _BN
        bt = (b / _BN)[None, :]
        if i == 0 and dpad is not None:
            wt = jnp.pad(wt, ((0, dpad - wt.shape[0]), (0, 0)))
        out.append((wt, bt))
    return out


def _prep_fc(layers):
    out = []
    for i, (w, b) in enumerate(layers):
        s = 1.0 if i == 2 else 1.0 / _BN
        out.append((jnp.transpose(w) * s, (b * s)[None, :]))
    return out


def kernel(pointcloud, params):
    pc = pointcloud
    x_b3n = pc[:, 0:3, :]
    x_3bn = jnp.transpose(x_b3n, (1, 0, 2))

    cx1, cy1, cz1 = _fps(x_3bn, _M1)
    c1 = jnp.stack([cx1, cy1, cz1], axis=-1).reshape(_B * _M1, 3)
    gidx1 = _ball_query(x_b3n, c1, 0.5, tm=64)
    table1 = jnp.transpose(pc, (0, 2, 1))
    table1 = jnp.pad(table1, ((0, 0), (0, 0), (0, 10))).reshape(_B * _N1, 16)
    g1 = _gather_rows(table1, gidx1.reshape(-1))
    c1pad = jnp.pad(c1, ((0, 0), (0, 13)))
    ws1 = _prep_sa(params['sa1'], 16)
    feats1 = _group_mlp(g1, c1pad, ws1, tm=64)

    x2_3bn = jnp.stack([cx1, cy1, cz1], axis=0)
    cx2, cy2, cz2 = _fps(x2_3bn, _M2)
    c2 = jnp.stack([cx2, cy2, cz2], axis=-1).reshape(_B * _M2, 3)
    x2_b3n = jnp.stack([cx1, cy1, cz1], axis=1)
    gidx2 = _ball_query(x2_b3n, c2, 1.0, tm=64)
    table2 = jnp.concatenate([c1, feats1], axis=1)
    table2 = jnp.pad(table2, ((0, 0), (0, 13)))
    g2 = _gather_rows(table2, gidx2.reshape(-1))
    c2pad = jnp.pad(c2, ((0, 0), (0, 141)))
    ws2 = _prep_sa(params['sa2'], 144)
    feats2 = _group_mlp(g2, c2pad, ws2, tm=16)

    t3 = jnp.concatenate([c2, feats2], axis=1)
    t3 = jnp.pad(t3, ((0, 0), (0, 13)))
    ws3 = _prep_sa(params['sa3'], 272)
    fcs = _prep_fc(params['fc'])
    return _head(t3, ws3, fcs)


# speedup vs baseline: 15.0918x; 1.3798x over previous
import functools
import jax
import jax.numpy as jnp
import numpy as np
from jax import lax
from jax.experimental import pallas as pl
from jax.experimental.pallas import tpu as pltpu
from jax.experimental.pallas import tpu_sc as plsc

_BN = float(np.sqrt(1.0 + 1e-5))
_B = 8
_N1 = 4096
_M1 = 512
_M2 = 128
_NS = 64


def _fps_body(x_ref, ox_ref, oy_ref, oz_ref, dists_ref, far_ref, *, n):
    t = pl.program_id(0)

    @pl.when(t == 0)
    def _():
        dists_ref[...] = jnp.full((_B, n), 1e10, jnp.float32)
        far_ref[...] = jnp.zeros((_B, 1), jnp.int32)

    far = far_ref[...]
    lane = jax.lax.broadcasted_iota(jnp.int32, (_B, n), 1)
    sel = lane == far
    xs = x_ref[0]
    ys = x_ref[1]
    zs = x_ref[2]
    cx = jnp.sum(jnp.where(sel, xs, 0.0), axis=1, keepdims=True)
    cy = jnp.sum(jnp.where(sel, ys, 0.0), axis=1, keepdims=True)
    cz = jnp.sum(jnp.where(sel, zs, 0.0), axis=1, keepdims=True)
    ox_ref[0] = cx
    oy_ref[0] = cy
    oz_ref[0] = cz
    d = (xs - cx) ** 2 + (ys - cy) ** 2 + (zs - cz) ** 2
    dmin = jnp.minimum(dists_ref[...], d)
    dists_ref[...] = dmin
    m = jnp.max(dmin, axis=1, keepdims=True)
    nxt = jnp.min(jnp.where(dmin == m, lane, n), axis=1, keepdims=True)
    far_ref[...] = nxt.astype(jnp.int32)


def _fps(x3bn, npoint):
    n = x3bn.shape[2]
    out = jax.ShapeDtypeStruct((npoint, _B, 1), jnp.float32)
    cx, cy, cz = pl.pallas_call(
        functools.partial(_fps_body, n=n),
        grid=(npoint,),
        in_specs=[pl.BlockSpec((3, _B, n), lambda t: (0, 0, 0))],
        out_specs=[pl.BlockSpec((1, _B, 1), lambda t: (t, 0, 0))] * 3,
        out_shape=[out, out, out],
        scratch_shapes=[pltpu.VMEM((_B, n), jnp.float32),
                        pltpu.VMEM((_B, 1), jnp.int32)],
    )(x3bn)
    tr = lambda a: jnp.transpose(a[:, :, 0], (1, 0))
    return tr(cx), tr(cy), tr(cz)


def _bq_body(x_ref, c_ref, o_ref, *, rsq, n, tm):
    b = pl.program_id(0)
    X = x_ref[0]
    C = c_ref[...]
    dot = jnp.dot(C, X, preferred_element_type=jnp.float32)
    a2 = jnp.sum(C * C, axis=1, keepdims=True)
    b2 = jnp.sum(X * X, axis=0, keepdims=True)
    sq = (-2.0 * dot) + a2 + b2
    mask = (sq <= rsq).astype(jnp.int32)
    pos = mask
    k = 1
    while k < n:
        shifted = jnp.concatenate(
            [jnp.zeros((tm, k), jnp.int32), pos[:, : n - k]], axis=1)
        pos = pos + shifted
        k *= 2
    cols = []
    for j in range(_NS):
        cols.append(jnp.sum((pos <= j).astype(jnp.int32), axis=1, keepdims=True))
    idx = jnp.concatenate(cols, axis=1)
    first = idx[:, 0:1]
    idx = jnp.where(idx == n, jnp.broadcast_to(first, idx.shape), idx)
    idx = jnp.where(idx == n, 0, idx)
    o_ref[...] = idx + b * n


def _ball_query(xb3n, cflat, radius, tm):
    n = xb3n.shape[2]
    m_total = cflat.shape[0]
    mper = m_total // _B
    grid = (_B, mper // tm)
    return pl.pallas_call(
        functools.partial(_bq_body, rsq=radius * radius, n=n, tm=tm),
        grid=grid,
        in_specs=[
            pl.BlockSpec((1, 3, n), lambda b, m: (b, 0, 0)),
            pl.BlockSpec((tm, 3), lambda b, m, mper_t=mper // tm: (b * mper_t + m, 0)),
        ],
        out_specs=pl.BlockSpec((tm, _NS), lambda b, m, mper_t=mper // tm: (b * mper_t + m, 0)),
        out_shape=jax.ShapeDtypeStruct((m_total, _NS), jnp.int32),
    )(xb3n, cflat)


def _mlp_body(g_ref, c_ref, w1_ref, b1_ref, w2_ref, b2_ref, w3_ref, b3_ref,
              o_ref, *, tm):
    rows = tm * _NS
    rep = (jax.lax.broadcasted_iota(jnp.int32, (rows, tm), 0) // _NS ==
           jax.lax.broadcasted_iota(jnp.int32, (rows, tm), 1)).astype(jnp.float32)
    ce = jnp.dot(rep, c_ref[...], preferred_element_type=jnp.float32)
    g = g_ref[...] - ce
    h = jax.nn.relu(jnp.dot(g, w1_ref[...], preferred_element_type=jnp.float32)
                    + b1_ref[...])
    h = jax.nn.relu(jnp.dot(h, w2_ref[...], preferred_element_type=jnp.float32)
                    + b2_ref[...])
    h = jax.nn.relu(jnp.dot(h, w3_ref[...], preferred_element_type=jnp.float32)
                    + b3_ref[...])
    for m in range(tm):
        o_ref[m:m + 1, :] = jnp.max(h[m * _NS:(m + 1) * _NS, :], axis=0,
                                    keepdims=True)


def _group_mlp(gather_rows, cpad, ws, tm):
    m_total = cpad.shape[0]
    dpad = gather_rows.shape[1]
    (w1, b1), (w2, b2), (w3, b3) = ws
    cout = w3.shape[1]
    grid = (m_total // tm,)
    full = lambda a: pl.BlockSpec(a.shape, lambda i: tuple(0 for _ in a.shape))
    return pl.pallas_call(
        functools.partial(_mlp_body, tm=tm),
        grid=grid,
        in_specs=[
            pl.BlockSpec((tm * _NS, dpad), lambda i: (i, 0)),
            pl.BlockSpec((tm, dpad), lambda i: (i, 0)),
            full(w1), full(b1), full(w2), full(b2), full(w3), full(b3),
        ],
        out_specs=pl.BlockSpec((tm, cout), lambda i: (i, 0)),
        out_shape=jax.ShapeDtypeStruct((m_total, cout), jnp.float32),
    )(gather_rows, cpad, w1, b1, w2, b2, w3, b3)


def _head_body(t_ref, w1_ref, b1_ref, w2_ref, b2_ref, w3_ref, b3_ref,
               f1_ref, fb1_ref, f2_ref, fb2_ref, f3_ref, fb3_ref,
               o_ref, pool_ref):
    h = t_ref[...]
    h = jax.nn.relu(jnp.dot(h, w1_ref[...], preferred_element_type=jnp.float32)
                    + b1_ref[...])
    h = jax.nn.relu(jnp.dot(h, w2_ref[...], preferred_element_type=jnp.float32)
                    + b2_ref[...])
    h = jax.nn.relu(jnp.dot(h, w3_ref[...], preferred_element_type=jnp.float32)
                    + b3_ref[...])
    for b in range(_B):
        pool_ref[b:b + 1, :] = jnp.max(h[b * _M2:(b + 1) * _M2, :], axis=0,
                                       keepdims=True)
    x = pool_ref[...]
    x = jax.nn.relu(jnp.dot(x, f1_ref[...], preferred_element_type=jnp.float32)
                    + fb1_ref[...])
    x = jax.nn.relu(jnp.dot(x, f2_ref[...], preferred_element_type=jnp.float32)
                    + fb2_ref[...])
    o_ref[...] = jnp.dot(x, f3_ref[...], preferred_element_type=jnp.float32) + fb3_ref[...]


def _head(t3, ws, fcs):
    (w1, b1), (w2, b2), (w3, b3) = ws
    (f1, fb1), (f2, fb2), (f3, fb3) = fcs
    nc = f3.shape[1]
    args = (t3, w1, b1, w2, b2, w3, b3, f1, fb1, f2, fb2, f3, fb3)
    return pl.pallas_call(
        _head_body,
        out_shape=jax.ShapeDtypeStruct((_B, nc), jnp.float32),
        scratch_shapes=[pltpu.VMEM((_B, w3.shape[1]), jnp.float32)],
    )(*args)


def _gather_rows(table, gidx):
    d = table.shape[1]
    r = gidx.shape[0]
    info = plsc.get_sparse_core_info()
    nw = info.num_cores * info.num_subcores
    chunk = 128
    b_per_w = r // nw
    nchunk = b_per_w // chunk
    assert b_per_w % chunk == 0 and r % (8 * nw) == 0
    mesh = plsc.VectorSubcoreMesh(core_axis_name="c", subcore_axis_name="s")

    @functools.partial(
        pl.kernel, mesh=mesh,
        out_type=jax.ShapeDtypeStruct((r, d), jnp.float32),
        scratch_types=[pltpu.VMEM((chunk,), jnp.int32),
                       pltpu.VMEM((chunk, d), jnp.float32),
                       pltpu.SemaphoreType.DMA],
    )
    def k(table_hbm, idx_hbm, out_hbm, idx_v, rows_v, sem):
        wid = lax.axis_index("s") * info.num_cores + lax.axis_index("c")
        base = wid * b_per_w

        def body(i, carry):
            off = base + i * chunk
            pltpu.sync_copy(idx_hbm.at[pl.ds(off, chunk)], idx_v)
            pltpu.async_copy(table_hbm.at[idx_v], rows_v, sem).wait()
            pltpu.sync_copy(rows_v, out_hbm.at[pl.ds(off, chunk)])
            return carry

        lax.fori_loop(0, nchunk, body, 0)

    return k(table, gidx)


def _prep_sa(layers, dpad):
    out = []
    for i, (w, b) in enumerate(layers):
        wt = jnp.transpose(w) / _BN
        bt = (b / _BN)[None, :]
        if i == 0 and dpad is not None:
            wt = jnp.pad(wt, ((0, dpad - wt.shape[0]), (0, 0)))
        out.append((wt, bt))
    return out


def _prep_fc(layers):
    out = []
    for i, (w, b) in enumerate(layers):
        s = 1.0 if i == 2 else 1.0 / _BN
        out.append((jnp.transpose(w) * s, (b * s)[None, :]))
    return out


def kernel(pointcloud, params):
    pc = pointcloud
    x_b3n = pc[:, 0:3, :]
    x_3bn = jnp.transpose(x_b3n, (1, 0, 2))

    cx1, cy1, cz1 = _fps(x_3bn, _M1)
    c1 = jnp.stack([cx1, cy1, cz1], axis=-1).reshape(_B * _M1, 3)
    gidx1 = _ball_query(x_b3n, c1, 0.5, tm=64)
    table1 = jnp.transpose(pc, (0, 2, 1))
    table1 = jnp.pad(table1, ((0, 0), (0, 0), (0, 122))).reshape(_B * _N1, 128)
    g1 = _gather_rows(table1, gidx1.reshape(-1))
    c1pad = jnp.pad(c1, ((0, 0), (0, 125)))
    ws1 = _prep_sa(params['sa1'], 128)
    feats1 = _group_mlp(g1, c1pad, ws1, tm=64)

    x2_3bn = jnp.stack([cx1, cy1, cz1], axis=0)
    cx2, cy2, cz2 = _fps(x2_3bn, _M2)
    c2 = jnp.stack([cx2, cy2, cz2], axis=-1).reshape(_B * _M2, 3)
    x2_b3n = jnp.stack([cx1, cy1, cz1], axis=1)
    gidx2 = _ball_query(x2_b3n, c2, 1.0, tm=64)
    table2 = jnp.concatenate([c1, feats1], axis=1)
    table2 = jnp.pad(table2, ((0, 0), (0, 125)))
    g2 = _gather_rows(table2, gidx2.reshape(-1))
    c2pad = jnp.pad(c2, ((0, 0), (0, 253)))
    ws2 = _prep_sa(params['sa2'], 256)
    feats2 = _group_mlp(g2, c2pad, ws2, tm=16)

    t3 = jnp.concatenate([c2, feats2], axis=1)
    t3 = jnp.pad(t3, ((0, 0), (0, 13)))
    ws3 = _prep_sa(params['sa3'], 272)
    fcs = _prep_fc(params['fc'])
    return _head(t3, ws3, fcs)


# speedup vs baseline: 16.9673x; 1.1243x over previous
import functools
import jax
import jax.numpy as jnp
import numpy as np
from jax import lax
from jax.experimental import pallas as pl
from jax.experimental.pallas import tpu as pltpu
from jax.experimental.pallas import tpu_sc as plsc

_BN = float(np.sqrt(1.0 + 1e-5))
_B = 8
_N1 = 4096
_M1 = 512
_M2 = 128
_NS = 64


def _fps_body(x_ref, ox_ref, oy_ref, oz_ref, dists_ref, far_ref, *, n):
    t = pl.program_id(0)

    @pl.when(t == 0)
    def _():
        dists_ref[...] = jnp.full((_B, n), 1e10, jnp.float32)
        far_ref[...] = jnp.zeros((_B, 1), jnp.int32)

    far = far_ref[...]
    lane = jax.lax.broadcasted_iota(jnp.int32, (_B, n), 1)
    sel = lane == far
    xs = x_ref[0]
    ys = x_ref[1]
    zs = x_ref[2]
    cx = jnp.sum(jnp.where(sel, xs, 0.0), axis=1, keepdims=True)
    cy = jnp.sum(jnp.where(sel, ys, 0.0), axis=1, keepdims=True)
    cz = jnp.sum(jnp.where(sel, zs, 0.0), axis=1, keepdims=True)
    ox_ref[0] = cx
    oy_ref[0] = cy
    oz_ref[0] = cz
    d = (xs - cx) ** 2 + (ys - cy) ** 2 + (zs - cz) ** 2
    dmin = jnp.minimum(dists_ref[...], d)
    dists_ref[...] = dmin
    m = jnp.max(dmin, axis=1, keepdims=True)
    nxt = jnp.min(jnp.where(dmin == m, lane, n), axis=1, keepdims=True)
    far_ref[...] = nxt.astype(jnp.int32)


def _fps(x3bn, npoint):
    n = x3bn.shape[2]
    out = jax.ShapeDtypeStruct((npoint, _B, 1), jnp.float32)
    cx, cy, cz = pl.pallas_call(
        functools.partial(_fps_body, n=n),
        grid=(npoint,),
        in_specs=[pl.BlockSpec((3, _B, n), lambda t: (0, 0, 0))],
        out_specs=[pl.BlockSpec((1, _B, 1), lambda t: (t, 0, 0))] * 3,
        out_shape=[out, out, out],
        scratch_shapes=[pltpu.VMEM((_B, n), jnp.float32),
                        pltpu.VMEM((_B, 1), jnp.int32)],
    )(x3bn)
    tr = lambda a: jnp.transpose(a[:, :, 0], (1, 0))
    return tr(cx), tr(cy), tr(cz)


def _bq_body(x_ref, c_ref, o_ref, *, rsq, n, tm):
    b = pl.program_id(0)
    X = x_ref[0]
    C = c_ref[...]
    dot = jnp.dot(C, X, preferred_element_type=jnp.float32)
    a2 = jnp.sum(C * C, axis=1, keepdims=True)
    b2 = jnp.sum(X * X, axis=0, keepdims=True)
    sq = (-2.0 * dot) + a2 + b2
    mask = (sq <= rsq).astype(jnp.int32)
    pos = mask
    k = 1
    while k < n:
        shifted = jnp.concatenate(
            [jnp.zeros((tm, k), jnp.int32), pos[:, : n - k]], axis=1)
        pos = pos + shifted
        k *= 2
    cols = []
    for j in range(_NS):
        cols.append(jnp.sum((pos <= j).astype(jnp.int32), axis=1, keepdims=True))
    idx = jnp.concatenate(cols, axis=1)
    first = idx[:, 0:1]
    idx = jnp.where(idx == n, jnp.broadcast_to(first, idx.shape), idx)
    idx = jnp.where(idx == n, 0, idx)
    o_ref[...] = idx + b * n


def _ball_query(xb3n, cflat, radius, tm):
    n = xb3n.shape[2]
    m_total = cflat.shape[0]
    mper = m_total // _B
    grid = (_B, mper // tm)
    return pl.pallas_call(
        functools.partial(_bq_body, rsq=radius * radius, n=n, tm=tm),
        grid=grid,
        in_specs=[
            pl.BlockSpec((1, 3, n), lambda b, m: (b, 0, 0)),
            pl.BlockSpec((tm, 3), lambda b, m, mper_t=mper // tm: (b * mper_t + m, 0)),
        ],
        out_specs=pl.BlockSpec((tm, _NS), lambda b, m, mper_t=mper // tm: (b * mper_t + m, 0)),
        out_shape=jax.ShapeDtypeStruct((m_total, _NS), jnp.int32),
    )(xb3n, cflat)


def _mlp_body(g_ref, c_ref, w1_ref, b1_ref, w2_ref, b2_ref, w3_ref, b3_ref,
              o_ref, *, tm):
    rows = tm * _NS
    rep = (jax.lax.broadcasted_iota(jnp.int32, (rows, tm), 0) // _NS ==
           jax.lax.broadcasted_iota(jnp.int32, (rows, tm), 1)).astype(jnp.float32)
    ce = jnp.dot(rep, c_ref[...], preferred_element_type=jnp.float32)
    g = g_ref[...] - ce
    h = jax.nn.relu(jnp.dot(g, w1_ref[...], preferred_element_type=jnp.float32)
                    + b1_ref[...])
    h = jax.nn.relu(jnp.dot(h, w2_ref[...], preferred_element_type=jnp.float32)
                    + b2_ref[...])
    h = jax.nn.relu(jnp.dot(h, w3_ref[...], preferred_element_type=jnp.float32)
                    + b3_ref[...])
    for m in range(tm):
        o_ref[m:m + 1, :] = jnp.max(h[m * _NS:(m + 1) * _NS, :], axis=0,
                                    keepdims=True)


def _group_mlp(gather_rows, cpad, ws, tm):
    m_total = cpad.shape[0]
    dpad = gather_rows.shape[1]
    (w1, b1), (w2, b2), (w3, b3) = ws
    cout = w3.shape[1]
    grid = (m_total // tm,)
    full = lambda a: pl.BlockSpec(a.shape, lambda i: tuple(0 for _ in a.shape))
    return pl.pallas_call(
        functools.partial(_mlp_body, tm=tm),
        grid=grid,
        in_specs=[
            pl.BlockSpec((tm * _NS, dpad), lambda i: (i, 0)),
            pl.BlockSpec((tm, dpad), lambda i: (i, 0)),
            full(w1), full(b1), full(w2), full(b2), full(w3), full(b3),
        ],
        out_specs=pl.BlockSpec((tm, cout), lambda i: (i, 0)),
        out_shape=jax.ShapeDtypeStruct((m_total, cout), jnp.float32),
    )(gather_rows, cpad, w1, b1, w2, b2, w3, b3)


def _head_body(t_ref, w1_ref, b1_ref, w2_ref, b2_ref, w3_ref, b3_ref,
               f1_ref, fb1_ref, f2_ref, fb2_ref, f3_ref, fb3_ref,
               o_ref, pool_ref):
    h = t_ref[...]
    h = jax.nn.relu(jnp.dot(h, w1_ref[...], preferred_element_type=jnp.float32)
                    + b1_ref[...])
    h = jax.nn.relu(jnp.dot(h, w2_ref[...], preferred_element_type=jnp.float32)
                    + b2_ref[...])
    h = jax.nn.relu(jnp.dot(h, w3_ref[...], preferred_element_type=jnp.float32)
                    + b3_ref[...])
    for b in range(_B):
        pool_ref[b:b + 1, :] = jnp.max(h[b * _M2:(b + 1) * _M2, :], axis=0,
                                       keepdims=True)
    x = pool_ref[...]
    x = jax.nn.relu(jnp.dot(x, f1_ref[...], preferred_element_type=jnp.float32)
                    + fb1_ref[...])
    x = jax.nn.relu(jnp.dot(x, f2_ref[...], preferred_element_type=jnp.float32)
                    + fb2_ref[...])
    o_ref[...] = jnp.dot(x, f3_ref[...], preferred_element_type=jnp.float32) + fb3_ref[...]


def _head(t3, ws, fcs):
    (w1, b1), (w2, b2), (w3, b3) = ws
    (f1, fb1), (f2, fb2), (f3, fb3) = fcs
    nc = f3.shape[1]
    args = (t3, w1, b1, w2, b2, w3, b3, f1, fb1, f2, fb2, f3, fb3)
    return pl.pallas_call(
        _head_body,
        out_shape=jax.ShapeDtypeStruct((_B, nc), jnp.float32),
        scratch_shapes=[pltpu.VMEM((_B, w3.shape[1]), jnp.float32)],
    )(*args)


def _gather_rows(table, gidx):
    d = table.shape[1]
    r = gidx.shape[0]
    info = plsc.get_sparse_core_info()
    nw = info.num_cores * info.num_subcores
    chunk = 128
    b_per_w = r // nw
    nchunk = b_per_w // chunk
    nbuf = 4 if chunk * d * 4 * 4 <= 300 * 1024 else 2
    assert b_per_w % chunk == 0 and nchunk % nbuf == 0 and r % (8 * nw) == 0
    mesh = plsc.VectorSubcoreMesh(core_axis_name="c", subcore_axis_name="s")

    @functools.partial(
        pl.kernel, mesh=mesh,
        out_type=jax.ShapeDtypeStruct((r, d), jnp.float32),
        scratch_types=[pltpu.VMEM((b_per_w,), jnp.int32)]
        + [pltpu.VMEM((chunk, d), jnp.float32)] * nbuf
        + [pltpu.SemaphoreType.DMA],
    )
    def k(table_hbm, idx_hbm, out_hbm, idx_v, *bufs_sem):
        rows_v = bufs_sem[:nbuf]
        sem = bufs_sem[nbuf]
        wid = lax.axis_index("s") * info.num_cores + lax.axis_index("c")
        base = wid * b_per_w
        pltpu.sync_copy(idx_hbm.at[pl.ds(base, b_per_w)], idx_v)

        def body(g, carry):
            cps = []
            for bi in range(nbuf):
                loc = (g * nbuf + bi) * chunk
                cps.append(pltpu.async_copy(
                    table_hbm.at[idx_v.at[pl.ds(loc, chunk)]], rows_v[bi], sem))
            for cp in cps:
                cp.wait()
            for bi in range(nbuf):
                loc = (g * nbuf + bi) * chunk
                pltpu.sync_copy(rows_v[bi], out_hbm.at[pl.ds(base + loc, chunk)])
            return carry

        lax.fori_loop(0, nchunk // nbuf, body, 0)

    return k(table, gidx)


def _prep_sa(layers, dpad):
    out = []
    for i, (w, b) in enumerate(layers):
        wt = jnp.transpose(w) / _BN
        bt = (b / _BN)[None, :]
        if i == 0 and dpad is not None:
            wt = jnp.pad(wt, ((0, dpad - wt.shape[0]), (0, 0)))
        out.append((wt, bt))
    return out


def _prep_fc(layers):
    out = []
    for i, (w, b) in enumerate(layers):
        s = 1.0 if i == 2 else 1.0 / _BN
        out.append((jnp.transpose(w) * s, (b * s)[None, :]))
    return out


def kernel(pointcloud, params):
    pc = pointcloud
    x_b3n = pc[:, 0:3, :]
    x_3bn = jnp.transpose(x_b3n, (1, 0, 2))

    cx1, cy1, cz1 = _fps(x_3bn, _M1)
    c1 = jnp.stack([cx1, cy1, cz1], axis=-1).reshape(_B * _M1, 3)
    gidx1 = _ball_query(x_b3n, c1, 0.5, tm=64)
    table1 = jnp.transpose(pc, (0, 2, 1))
    table1 = jnp.pad(table1, ((0, 0), (0, 0), (0, 122))).reshape(_B * _N1, 128)
    g1 = _gather_rows(table1, gidx1.reshape(-1))
    c1pad = jnp.pad(c1, ((0, 0), (0, 125)))
    ws1 = _prep_sa(params['sa1'], 128)
    feats1 = _group_mlp(g1, c1pad, ws1, tm=64)

    x2_3bn = jnp.stack([cx1, cy1, cz1], axis=0)
    cx2, cy2, cz2 = _fps(x2_3bn, _M2)
    c2 = jnp.stack([cx2, cy2, cz2], axis=-1).reshape(_B * _M2, 3)
    x2_b3n = jnp.stack([cx1, cy1, cz1], axis=1)
    gidx2 = _ball_query(x2_b3n, c2, 1.0, tm=64)
    table2 = jnp.concatenate([c1, feats1], axis=1)
    table2 = jnp.pad(table2, ((0, 0), (0, 125)))
    g2 = _gather_rows(table2, gidx2.reshape(-1))
    c2pad = jnp.pad(c2, ((0, 0), (0, 253)))
    ws2 = _prep_sa(params['sa2'], 256)
    feats2 = _group_mlp(g2, c2pad, ws2, tm=16)

    t3 = jnp.concatenate([c2, feats2], axis=1)
    t3 = jnp.pad(t3, ((0, 0), (0, 13)))
    ws3 = _prep_sa(params['sa3'], 272)
    fcs = _prep_fc(params['fc'])
    return _head(t3, ws3, fcs)


# speedup vs baseline: 18.5051x; 1.0906x over previous
import functools
import jax
import jax.numpy as jnp
import numpy as np
from jax import lax
from jax.experimental import pallas as pl
from jax.experimental.pallas import tpu as pltpu
from jax.experimental.pallas import tpu_sc as plsc

_BN = float(np.sqrt(1.0 + 1e-5))
_B = 8
_N1 = 4096
_M1 = 512
_M2 = 128
_NS = 64


def _fps_body(x_ref, ox_ref, oy_ref, oz_ref, dists_ref, far_ref, *, n):
    t = pl.program_id(0)

    @pl.when(t == 0)
    def _():
        dists_ref[...] = jnp.full((_B, n), 1e10, jnp.float32)
        far_ref[...] = jnp.zeros((_B, 1), jnp.int32)

    far = far_ref[...]
    lane = jax.lax.broadcasted_iota(jnp.int32, (_B, n), 1)
    sel = lane == far
    xs = x_ref[0]
    ys = x_ref[1]
    zs = x_ref[2]
    cx = jnp.sum(jnp.where(sel, xs, 0.0), axis=1, keepdims=True)
    cy = jnp.sum(jnp.where(sel, ys, 0.0), axis=1, keepdims=True)
    cz = jnp.sum(jnp.where(sel, zs, 0.0), axis=1, keepdims=True)
    ox_ref[0] = cx
    oy_ref[0] = cy
    oz_ref[0] = cz
    d = (xs - cx) ** 2 + (ys - cy) ** 2 + (zs - cz) ** 2
    dmin = jnp.minimum(dists_ref[...], d)
    dists_ref[...] = dmin
    m = jnp.max(dmin, axis=1, keepdims=True)
    nxt = jnp.min(jnp.where(dmin == m, lane, n), axis=1, keepdims=True)
    far_ref[...] = nxt.astype(jnp.int32)


def _fps(x3bn, npoint):
    n = x3bn.shape[2]
    out = jax.ShapeDtypeStruct((npoint, _B, 1), jnp.float32)
    cx, cy, cz = pl.pallas_call(
        functools.partial(_fps_body, n=n),
        grid=(npoint,),
        in_specs=[pl.BlockSpec((3, _B, n), lambda t: (0, 0, 0))],
        out_specs=[pl.BlockSpec((1, _B, 1), lambda t: (t, 0, 0))] * 3,
        out_shape=[out, out, out],
        scratch_shapes=[pltpu.VMEM((_B, n), jnp.float32),
                        pltpu.VMEM((_B, 1), jnp.int32)],
    )(x3bn)
    tr = lambda a: jnp.transpose(a[:, :, 0], (1, 0))
    return tr(cx), tr(cy), tr(cz)


def _bq_body(x_ref, c_ref, o_ref, *, rsq, n, tm):
    b = pl.program_id(0)
    X = x_ref[0]
    C = c_ref[...]
    dot = jnp.dot(C, X, preferred_element_type=jnp.float32)
    a2 = jnp.sum(C * C, axis=1, keepdims=True)
    b2 = jnp.sum(X * X, axis=0, keepdims=True)
    sq = (-2.0 * dot) + a2 + b2
    mask = (sq <= rsq).astype(jnp.int32)
    pos = mask
    k = 1
    while k < n:
        shifted = jnp.concatenate(
            [jnp.zeros((tm, k), jnp.int32), pos[:, : n - k]], axis=1)
        pos = pos + shifted
        k *= 2
    cols = []
    for j in range(_NS):
        cols.append(jnp.sum((pos <= j).astype(jnp.int32), axis=1, keepdims=True))
    idx = jnp.concatenate(cols, axis=1)
    first = idx[:, 0:1]
    idx = jnp.where(idx == n, jnp.broadcast_to(first, idx.shape), idx)
    idx = jnp.where(idx == n, 0, idx)
    o_ref[...] = idx + b * n


def _ball_query(xb3n, cflat, radius, tm):
    n = xb3n.shape[2]
    m_total = cflat.shape[0]
    mper = m_total // _B
    grid = (_B, mper // tm)
    return pl.pallas_call(
        functools.partial(_bq_body, rsq=radius * radius, n=n, tm=tm),
        grid=grid,
        in_specs=[
            pl.BlockSpec((1, 3, n), lambda b, m: (b, 0, 0)),
            pl.BlockSpec((tm, 3), lambda b, m, mper_t=mper // tm: (b * mper_t + m, 0)),
        ],
        out_specs=pl.BlockSpec((tm, _NS), lambda b, m, mper_t=mper // tm: (b * mper_t + m, 0)),
        out_shape=jax.ShapeDtypeStruct((m_total, _NS), jnp.int32),
    )(xb3n, cflat)


def _mlp_body(g_ref, c_ref, w1_ref, b1_ref, w2_ref, b2_ref, w3_ref, b3_ref,
              o_ref, *, tm):
    rows = tm * _NS
    rep = (jax.lax.broadcasted_iota(jnp.int32, (rows, tm), 0) // _NS ==
           jax.lax.broadcasted_iota(jnp.int32, (rows, tm), 1)).astype(jnp.float32)
    ce = jnp.dot(rep, c_ref[...], preferred_element_type=jnp.float32)
    g = g_ref[...] - ce
    h = jax.nn.relu(jnp.dot(g, w1_ref[...], preferred_element_type=jnp.float32)
                    + b1_ref[...])
    h = jax.nn.relu(jnp.dot(h, w2_ref[...], preferred_element_type=jnp.float32)
                    + b2_ref[...])
    h = jax.nn.relu(jnp.dot(h, w3_ref[...], preferred_element_type=jnp.float32)
                    + b3_ref[...])
    for m in range(tm):
        o_ref[m:m + 1, :] = jnp.max(h[m * _NS:(m + 1) * _NS, :], axis=0,
                                    keepdims=True)


def _group_mlp(gather_rows, cpad, ws, tm):
    m_total = cpad.shape[0]
    dpad = gather_rows.shape[1]
    (w1, b1), (w2, b2), (w3, b3) = ws
    cout = w3.shape[1]
    grid = (m_total // tm,)
    full = lambda a: pl.BlockSpec(a.shape, lambda i: tuple(0 for _ in a.shape))
    return pl.pallas_call(
        functools.partial(_mlp_body, tm=tm),
        grid=grid,
        in_specs=[
            pl.BlockSpec((tm * _NS, dpad), lambda i: (i, 0)),
            pl.BlockSpec((tm, dpad), lambda i: (i, 0)),
            full(w1), full(b1), full(w2), full(b2), full(w3), full(b3),
        ],
        out_specs=pl.BlockSpec((tm, cout), lambda i: (i, 0)),
        out_shape=jax.ShapeDtypeStruct((m_total, cout), jnp.float32),
    )(gather_rows, cpad, w1, b1, w2, b2, w3, b3)


def _head_body(t_ref, w1_ref, b1_ref, w2_ref, b2_ref, w3_ref, b3_ref,
               f1_ref, fb1_ref, f2_ref, fb2_ref, f3_ref, fb3_ref,
               o_ref, pool_ref):
    h = t_ref[...]
    h = jax.nn.relu(jnp.dot(h, w1_ref[...], preferred_element_type=jnp.float32)
                    + b1_ref[...])
    h = jax.nn.relu(jnp.dot(h, w2_ref[...], preferred_element_type=jnp.float32)
                    + b2_ref[...])
    h = jax.nn.relu(jnp.dot(h, w3_ref[...], preferred_element_type=jnp.float32)
                    + b3_ref[...])
    for b in range(_B):
        pool_ref[b:b + 1, :] = jnp.max(h[b * _M2:(b + 1) * _M2, :], axis=0,
                                       keepdims=True)
    x = pool_ref[...]
    x = jax.nn.relu(jnp.dot(x, f1_ref[...], preferred_element_type=jnp.float32)
                    + fb1_ref[...])
    x = jax.nn.relu(jnp.dot(x, f2_ref[...], preferred_element_type=jnp.float32)
                    + fb2_ref[...])
    o_ref[...] = jnp.dot(x, f3_ref[...], preferred_element_type=jnp.float32) + fb3_ref[...]


def _head(t3, ws, fcs):
    (w1, b1), (w2, b2), (w3, b3) = ws
    (f1, fb1), (f2, fb2), (f3, fb3) = fcs
    nc = f3.shape[1]
    args = (t3, w1, b1, w2, b2, w3, b3, f1, fb1, f2, fb2, f3, fb3)
    return pl.pallas_call(
        _head_body,
        out_shape=jax.ShapeDtypeStruct((_B, nc), jnp.float32),
        scratch_shapes=[pltpu.VMEM((_B, w3.shape[1]), jnp.float32)],
    )(*args)


def _gather_rows(table, gidx):
    d = table.shape[1]
    r = gidx.shape[0]
    info = plsc.get_sparse_core_info()
    nw = info.num_cores * info.num_subcores
    chunk = 128
    b_per_w = r // nw
    nchunk = b_per_w // chunk
    nbuf = 4 if chunk * d * 4 * 4 <= 300 * 1024 else 2
    assert b_per_w % chunk == 0 and nchunk % nbuf == 0 and r % (8 * nw) == 0
    mesh = plsc.VectorSubcoreMesh(core_axis_name="c", subcore_axis_name="s")

    @functools.partial(
        pl.kernel, mesh=mesh,
        compiler_params=pltpu.CompilerParams(use_tc_tiling_on_sc=False),
        out_type=jax.ShapeDtypeStruct((r, d), jnp.float32),
        scratch_types=[pltpu.VMEM((b_per_w,), jnp.int32)]
        + [pltpu.VMEM((chunk, d), jnp.float32)] * nbuf
        + [pltpu.SemaphoreType.DMA],
    )
    def k(table_hbm, idx_hbm, out_hbm, idx_v, *bufs_sem):
        rows_v = bufs_sem[:nbuf]
        sem = bufs_sem[nbuf]
        wid = lax.axis_index("s") * info.num_cores + lax.axis_index("c")
        base = wid * b_per_w
        pltpu.sync_copy(idx_hbm.at[pl.ds(base, b_per_w)], idx_v)

        def body(g, carry):
            cps = []
            for bi in range(nbuf):
                loc = (g * nbuf + bi) * chunk
                cps.append(pltpu.async_copy(
                    table_hbm.at[idx_v.at[pl.ds(loc, chunk)]], rows_v[bi], sem))
            for cp in cps:
                cp.wait()
            for bi in range(nbuf):
                loc = (g * nbuf + bi) * chunk
                pltpu.sync_copy(rows_v[bi], out_hbm.at[pl.ds(base + loc, chunk)])
            return carry

        lax.fori_loop(0, nchunk // nbuf, body, 0)

    return k(table, gidx)


def _prep_sa(layers, dpad):
    out = []
    for i, (w, b) in enumerate(layers):
        wt = jnp.transpose(w) / _BN
        bt = (b / _BN)[None, :]
        if i == 0 and dpad is not None:
            wt = jnp.pad(wt, ((0, dpad - wt.shape[0]), (0, 0)))
        out.append((wt, bt))
    return out


def _prep_fc(layers):
    out = []
    for i, (w, b) in enumerate(layers):
        s = 1.0 if i == 2 else 1.0 / _BN
        out.append((jnp.transpose(w) * s, (b * s)[None, :]))
    return out


def kernel(pointcloud, params):
    pc = pointcloud
    x_b3n = pc[:, 0:3, :]
    x_3bn = jnp.transpose(x_b3n, (1, 0, 2))

    cx1, cy1, cz1 = _fps(x_3bn, _M1)
    c1 = jnp.stack([cx1, cy1, cz1], axis=-1).reshape(_B * _M1, 3)
    gidx1 = _ball_query(x_b3n, c1, 0.5, tm=64)
    table1 = jnp.transpose(pc, (0, 2, 1))
    table1 = jnp.pad(table1, ((0, 0), (0, 0), (0, 10))).reshape(_B * _N1, 16)
    g1 = _gather_rows(table1, gidx1.reshape(-1))
    c1pad = jnp.pad(c1, ((0, 0), (0, 13)))
    ws1 = _prep_sa(params['sa1'], 16)
    feats1 = _group_mlp(g1, c1pad, ws1, tm=64)

    x2_3bn = jnp.stack([cx1, cy1, cz1], axis=0)
    cx2, cy2, cz2 = _fps(x2_3bn, _M2)
    c2 = jnp.stack([cx2, cy2, cz2], axis=-1).reshape(_B * _M2, 3)
    x2_b3n = jnp.stack([cx1, cy1, cz1], axis=1)
    gidx2 = _ball_query(x2_b3n, c2, 1.0, tm=64)
    table2 = jnp.concatenate([c1, feats1], axis=1)
    table2 = jnp.pad(table2, ((0, 0), (0, 13)))
    g2 = _gather_rows(table2, gidx2.reshape(-1))
    c2pad = jnp.pad(c2, ((0, 0), (0, 141)))
    ws2 = _prep_sa(params['sa2'], 144)
    feats2 = _group_mlp(g2, c2pad, ws2, tm=16)

    t3 = jnp.concatenate([c2, feats2], axis=1)
    t3 = jnp.pad(t3, ((0, 0), (0, 13)))
    ws3 = _prep_sa(params['sa3'], 272)
    fcs = _prep_fc(params['fc'])
    return _head(t3, ws3, fcs)
